# baseline (speedup 1.0000x reference)
import jax
import jax.numpy as jnp
from jax import lax
from jax.experimental import pallas as pl
from jax.experimental.pallas import tpu as pltpu

_N = 32768
_B = 16
_F = 128
_NP = 128
_NQ = 64
_IT = 4
_KEY = 1024.0
_LOGN = 15
_CHN = 4096
_LOGC = 12


def _prep_body(b_ref, cw_ref, idx_ref):
    bv = b_ref[...]
    cw = cw_ref[...]
    off = jnp.int32(0)
    for b in range(_B):
        cnt = jnp.sum((bv == b).astype(jnp.int32))
        cf = (cnt - 1).astype(jnp.float32)
        q = jnp.floor(cw * cf).astype(jnp.int32)
        row = off + q
        row = jnp.where(row < 0, row + _N, row)
        row = jnp.clip(row, 0, _N - 1)
        idx_ref[b, :] = row[0]
        off = off + cnt


def _matmul_body(x_ref, p_ref, b_ref, y_ref):
    acc = jnp.dot(x_ref[...], p_ref[...], preferred_element_type=jnp.float32)
    key = b_ref[...].astype(jnp.float32) * _KEY
    y_ref[0] = acc + key


def _sort_body(y_hbm, idx_ref, out_ref, scr_ref, aux_ref, sem):
    cp = pltpu.make_async_copy(y_hbm.at[pl.program_id(0)], scr_ref, sem)
    cp.start()
    cp.wait()

    def stage(kk, j):
        d = jnp.int32(1) << j

        @pl.when(j < _LOGC)
        def _small():
            for ch in range(_N // _CHN):
                sl = pl.ds(ch * _CHN, _CHN)
                i = lax.broadcasted_iota(jnp.int32, (_CHN, 1), 0) + ch * _CHN
                bit_j = (i >> j) & 1
                bit_kk = (i >> kk) & 1
                yc = scr_ref[sl, :]
                up = pltpu.roll(yc, _CHN - d, 0)
                dn = pltpu.roll(yc, d, 0)
                partner = jnp.where(bit_j == 0, up, dn)
                take_min = bit_j == bit_kk
                scr_ref[sl, :] = jnp.where(
                    take_min, jnp.minimum(yc, partner), jnp.maximum(yc, partner)
                )

        @pl.when(j >= _LOGC)
        def _big():
            for ch in range(_N // _CHN):
                a = ch * _CHN
                bit_j = (jnp.int32(a) >> j) & 1
                bit_kk = (jnp.int32(a) >> kk) & 1
                start = jnp.where(bit_j == 0, a + d, a - d)
                yc = scr_ref[pl.ds(a, _CHN), :]
                partner = scr_ref[pl.ds(start, _CHN), :]
                take_min = bit_j == bit_kk
                aux_ref[pl.ds(a, _CHN), :] = jnp.where(
                    take_min, jnp.minimum(yc, partner), jnp.maximum(yc, partner)
                )
            for ch in range(_N // _CHN):
                sl = pl.ds(ch * _CHN, _CHN)
                scr_ref[sl, :] = aux_ref[sl, :]

    def outer(kk, carry):
        def inner(jj, c2):
            stage(kk, kk - 1 - jj)
            return c2

        return lax.fori_loop(0, kk, inner, carry)

    lax.fori_loop(1, _LOGN + 1, outer, 0)

    inv = 1.0 / float((_NQ * _NP) ** 0.5)

    def gb(b, carry):
        keyoff = b.astype(jnp.float32) * _KEY

        def gq(q, c2):
            r = idx_ref[b, q]
            row = scr_ref[pl.ds(r, 1), :]
            out_ref[0, pl.ds(b * _NQ + q, 1), :] = (row - keyoff) * inv
            return c2

        return lax.fori_loop(0, _NQ, gq, carry)

    lax.fori_loop(0, _B, gb, 0)


@jax.jit
def kernel(x, batch, projections, cum_weights):
    batch32 = batch.astype(jnp.int32)

    row_idx = pl.pallas_call(
        _prep_body,
        grid=(1,),
        in_specs=[
            pl.BlockSpec((_N // 128, 128), lambda g: (0, 0)),
            pl.BlockSpec((1, _NQ), lambda g: (0, 0)),
        ],
        out_specs=pl.BlockSpec((_B, _NQ), lambda g: (0, 0)),
        out_shape=jax.ShapeDtypeStruct((_B, _NQ), jnp.int32),
    )(batch32.reshape(_N // 128, 128), cum_weights.reshape(1, _NQ))

    rb = 4096
    y = pl.pallas_call(
        _matmul_body,
        grid=(_IT, _N // rb),
        in_specs=[
            pl.BlockSpec((rb, _F), lambda i, r: (r, i)),
            pl.BlockSpec((_F, _NP), lambda i, r: (0, 0)),
            pl.BlockSpec((rb, 1), lambda i, r: (r, 0)),
        ],
        out_specs=pl.BlockSpec((1, rb, _NP), lambda i, r: (i, r, 0)),
        out_shape=jax.ShapeDtypeStruct((_IT, _N, _NP), jnp.float32),
    )(x, projections, batch32.reshape(_N, 1))

    q = pl.pallas_call(
        _sort_body,
        grid=(_IT,),
        in_specs=[
            pl.BlockSpec(memory_space=pl.ANY),
            pl.BlockSpec(memory_space=pltpu.SMEM),
        ],
        out_specs=pl.BlockSpec((1, _B * _NQ, _NP), lambda i: (i, 0, 0)),
        out_shape=jax.ShapeDtypeStruct((_IT, _B * _NQ, _NP), jnp.float32),
        scratch_shapes=[
            pltpu.VMEM((_N, _NP), jnp.float32),
            pltpu.VMEM((_N, _NP), jnp.float32),
            pltpu.SemaphoreType.DMA,
        ],
    )(y, row_idx)

    out = q.reshape(_IT, _B, _NQ, _NP).transpose(1, 0, 2, 3)
    return out.reshape(_B, _IT * _NQ * _NP)


# speedup vs baseline: 8.8281x; 8.8281x over previous
import jax
import jax.numpy as jnp
from jax import lax
from jax.experimental import pallas as pl
from jax.experimental.pallas import tpu as pltpu

_N = 32768
_B = 16
_F = 128
_NP = 128
_NQ = 64
_IT = 4
_KEY = 1024.0
_LOGN = 15
_CHN = 512
_LOGC = 9


def _prep_body(b_ref, cw_ref, idx_ref):
    bv = b_ref[...]
    cw = cw_ref[...]
    off = jnp.int32(0)
    for b in range(_B):
        cnt = jnp.sum((bv == b).astype(jnp.int32))
        cf = (cnt - 1).astype(jnp.float32)
        q = jnp.floor(cw * cf).astype(jnp.int32)
        row = off + q
        row = jnp.where(row < 0, row + _N, row)
        row = jnp.clip(row, 0, _N - 1)
        idx_ref[b, :] = row[0]
        off = off + cnt


def _matmul_body(x_ref, p_ref, b_ref, y_ref):
    acc = jnp.dot(x_ref[...], p_ref[...], preferred_element_type=jnp.float32)
    key = b_ref[...].astype(jnp.float32) * _KEY
    y_ref[0] = acc + key


def _sort_body(y_hbm, idx_ref, out_ref, scr_ref, sem):
    cp = pltpu.make_async_copy(y_hbm.at[pl.program_id(0)], scr_ref, sem)
    cp.start()
    cp.wait()

    nch = _N // _CHN
    logc = _LOGC

    def _cx(yc, r, j, bit_kk):
        d = 1 << j
        up = pltpu.roll(yc, _CHN - d, 0)
        dn = pltpu.roll(yc, d, 0)
        bit_j = (r >> j) & 1
        partner = jnp.where(bit_j == 0, up, dn)
        take_min = bit_j == bit_kk
        return jnp.where(
            take_min, jnp.minimum(yc, partner), jnp.maximum(yc, partner)
        )

    def chunk_sort(ch, carry):
        a = ch * _CHN
        r = lax.broadcasted_iota(jnp.int32, (_CHN, 1), 0)
        yc = scr_ref[pl.ds(a, _CHN), :]
        par = ch & 1
        for kk in range(1, logc + 1):
            for j in range(kk - 1, -1, -1):
                bit_kk = ((r >> kk) & 1) if kk < logc else par
                yc = _cx(yc, r, j, bit_kk)
        scr_ref[pl.ds(a, _CHN), :] = yc
        return carry

    lax.fori_loop(0, nch, chunk_sort, 0)

    for kk in range(logc + 1, _LOGN + 1):
        for j in range(kk - 1, logc - 1, -1):
            d = 1 << j
            jc = j - logc

            def big(t, carry, kk=kk, d=d, jc=jc):
                low = t & ((1 << jc) - 1)
                ca = ((t >> jc) << (jc + 1)) | low
                a_lo = ca * _CHN
                a_hi = a_lo + d
                ya = scr_ref[pl.ds(a_lo, _CHN), :]
                yb = scr_ref[pl.ds(a_hi, _CHN), :]
                asc = ((a_lo >> kk) & 1) == 0
                mn = jnp.minimum(ya, yb)
                mx = jnp.maximum(ya, yb)
                scr_ref[pl.ds(a_lo, _CHN), :] = jnp.where(asc, mn, mx)
                scr_ref[pl.ds(a_hi, _CHN), :] = jnp.where(asc, mx, mn)
                return carry

            lax.fori_loop(0, nch // 2, big, 0)

        def merge_chunk(ch, carry, kk=kk):
            a = ch * _CHN
            r = lax.broadcasted_iota(jnp.int32, (_CHN, 1), 0)
            yc = scr_ref[pl.ds(a, _CHN), :]
            bit_kk = (a >> kk) & 1
            for j in range(logc - 1, -1, -1):
                yc = _cx(yc, r, j, bit_kk)
            scr_ref[pl.ds(a, _CHN), :] = yc
            return carry

        lax.fori_loop(0, nch, merge_chunk, 0)

    inv = 1.0 / float((_NQ * _NP) ** 0.5)

    def gb(b, carry):
        keyoff = b.astype(jnp.float32) * _KEY

        def gq(q, c2):
            r = idx_ref[b, q]
            row = scr_ref[pl.ds(r, 1), :]
            out_ref[0, pl.ds(b * _NQ + q, 1), :] = (row - keyoff) * inv
            return c2

        return lax.fori_loop(0, _NQ, gq, carry)

    lax.fori_loop(0, _B, gb, 0)


@jax.jit
def kernel(x, batch, projections, cum_weights):
    batch32 = batch.astype(jnp.int32)

    row_idx = pl.pallas_call(
        _prep_body,
        grid=(1,),
        in_specs=[
            pl.BlockSpec((_N // 128, 128), lambda g: (0, 0)),
            pl.BlockSpec((1, _NQ), lambda g: (0, 0)),
        ],
        out_specs=pl.BlockSpec((_B, _NQ), lambda g: (0, 0)),
        out_shape=jax.ShapeDtypeStruct((_B, _NQ), jnp.int32),
    )(batch32.reshape(_N // 128, 128), cum_weights.reshape(1, _NQ))

    rb = 4096
    y = pl.pallas_call(
        _matmul_body,
        grid=(_IT, _N // rb),
        in_specs=[
            pl.BlockSpec((rb, _F), lambda i, r: (r, i)),
            pl.BlockSpec((_F, _NP), lambda i, r: (0, 0)),
            pl.BlockSpec((rb, 1), lambda i, r: (r, 0)),
        ],
        out_specs=pl.BlockSpec((1, rb, _NP), lambda i, r: (i, r, 0)),
        out_shape=jax.ShapeDtypeStruct((_IT, _N, _NP), jnp.float32),
    )(x, projections, batch32.reshape(_N, 1))

    q = pl.pallas_call(
        _sort_body,
        grid=(_IT,),
        in_specs=[
            pl.BlockSpec(memory_space=pl.ANY),
            pl.BlockSpec(memory_space=pltpu.SMEM),
        ],
        out_specs=pl.BlockSpec((1, _B * _NQ, _NP), lambda i: (i, 0, 0)),
        out_shape=jax.ShapeDtypeStruct((_IT, _B * _NQ, _NP), jnp.float32),
        scratch_shapes=[
            pltpu.VMEM((_N, _NP), jnp.float32),
            pltpu.SemaphoreType.DMA,
        ],
    )(y, row_idx)

    out = q.reshape(_IT, _B, _NQ, _NP).transpose(1, 0, 2, 3)
    return out.reshape(_B, _IT * _NQ * _NP)


# speedup vs baseline: 8.9469x; 1.0135x over previous
import jax
import jax.numpy as jnp
from jax import lax
from jax.experimental import pallas as pl
from jax.experimental.pallas import tpu as pltpu

_N = 32768
_B = 16
_F = 128
_NP = 128
_NQ = 64
_IT = 4
_KEY = 1024.0
_LOGN = 15
_CHN = 512
_LOGC = 9


def _prep_body(b_ref, cw_ref, idx_ref):
    bv = b_ref[...]
    cw = cw_ref[...]
    off = jnp.int32(0)
    for b in range(_B):
        cnt = jnp.sum((bv == b).astype(jnp.int32))
        cf = (cnt - 1).astype(jnp.float32)
        q = jnp.floor(cw * cf).astype(jnp.int32)
        row = off + q
        row = jnp.where(row < 0, row + _N, row)
        row = jnp.clip(row, 0, _N - 1)
        idx_ref[b, :] = row[0]
        off = off + cnt


def _matmul_body(x_ref, p_ref, b_ref, y_ref):
    acc = jnp.dot(x_ref[...], p_ref[...], preferred_element_type=jnp.float32)
    key = b_ref[...].astype(jnp.float32) * _KEY
    y_ref[0] = acc + key


def _sort_body(y_hbm, idx_ref, out_ref, scr_ref, sem):
    cp = pltpu.make_async_copy(y_hbm.at[pl.program_id(0)], scr_ref, sem)
    cp.start()
    cp.wait()

    nch = _N // _CHN
    logc = _LOGC

    def _cx(yc, r, j, bit_kk):
        d = 1 << j
        up = pltpu.roll(yc, _CHN - d, 0)
        dn = pltpu.roll(yc, d, 0)
        bit_j = (r >> j) & 1
        partner = jnp.where(bit_j == 0, up, dn)
        take_min = bit_j == bit_kk
        return jnp.where(
            take_min, jnp.minimum(yc, partner), jnp.maximum(yc, partner)
        )

    def chunk_sort(ch, carry):
        a = ch * _CHN
        r = lax.broadcasted_iota(jnp.int32, (_CHN, 1), 0)
        yc = scr_ref[pl.ds(a, _CHN), :]
        par = ch & 1
        for kk in range(1, logc + 1):
            for j in range(kk - 1, -1, -1):
                bit_kk = ((r >> kk) & 1) if kk < logc else par
                yc = _cx(yc, r, j, bit_kk)
        scr_ref[pl.ds(a, _CHN), :] = yc
        return carry

    lax.fori_loop(0, nch, chunk_sort, 0)

    def _pair_cx(ya, yb, asc):
        mn = jnp.minimum(ya, yb)
        mx = jnp.maximum(ya, yb)
        return jnp.where(asc, mn, mx), jnp.where(asc, mx, mn)

    for kk in range(logc + 1, _LOGN + 1):
        js = list(range(kk - 1, logc - 1, -1))
        fused = []
        while js:
            if len(js) >= 2:
                fused.append((js[0], js[1]))
                js = js[2:]
            else:
                fused.append((js[0],))
                js = js[1:]
        for grp in fused:
            if len(grp) == 1:
                (j,) = grp
                d = 1 << j
                jc = j - logc

                def big(t, carry, kk=kk, d=d, jc=jc):
                    low = t & ((1 << jc) - 1)
                    ca = ((t >> jc) << (jc + 1)) | low
                    a_lo = ca * _CHN
                    a_hi = a_lo + d
                    ya = scr_ref[pl.ds(a_lo, _CHN), :]
                    yb = scr_ref[pl.ds(a_hi, _CHN), :]
                    asc = ((a_lo >> kk) & 1) == 0
                    na, nb = _pair_cx(ya, yb, asc)
                    scr_ref[pl.ds(a_lo, _CHN), :] = na
                    scr_ref[pl.ds(a_hi, _CHN), :] = nb
                    return carry

                lax.fori_loop(0, nch // 2, big, 0)
            else:
                j1, j2 = grp
                d1, d2 = 1 << j1, 1 << j2
                jc2 = j2 - logc

                def big2(t, carry, kk=kk, d1=d1, d2=d2, jc2=jc2):
                    low = t & ((1 << jc2) - 1)
                    ca = ((t >> jc2) << (jc2 + 2)) | low
                    a0 = ca * _CHN
                    asc = ((a0 >> kk) & 1) == 0
                    ya = scr_ref[pl.ds(a0, _CHN), :]
                    yb = scr_ref[pl.ds(a0 + d2, _CHN), :]
                    yc = scr_ref[pl.ds(a0 + d1, _CHN), :]
                    yd = scr_ref[pl.ds(a0 + d1 + d2, _CHN), :]
                    ya, yc = _pair_cx(ya, yc, asc)
                    yb, yd = _pair_cx(yb, yd, asc)
                    ya, yb = _pair_cx(ya, yb, asc)
                    yc, yd = _pair_cx(yc, yd, asc)
                    scr_ref[pl.ds(a0, _CHN), :] = ya
                    scr_ref[pl.ds(a0 + d2, _CHN), :] = yb
                    scr_ref[pl.ds(a0 + d1, _CHN), :] = yc
                    scr_ref[pl.ds(a0 + d1 + d2, _CHN), :] = yd
                    return carry

                lax.fori_loop(0, nch // 4, big2, 0)

        def merge_chunk(ch, carry, kk=kk):
            a = ch * _CHN
            r = lax.broadcasted_iota(jnp.int32, (_CHN, 1), 0)
            yc = scr_ref[pl.ds(a, _CHN), :]
            bit_kk = (a >> kk) & 1
            for j in range(logc - 1, -1, -1):
                yc = _cx(yc, r, j, bit_kk)
            scr_ref[pl.ds(a, _CHN), :] = yc
            return carry

        lax.fori_loop(0, nch, merge_chunk, 0)

    inv = 1.0 / float((_NQ * _NP) ** 0.5)

    def gb(b, carry):
        keyoff = b.astype(jnp.float32) * _KEY

        def gq(q, c2):
            r = idx_ref[b, q]
            row = scr_ref[pl.ds(r, 1), :]
            out_ref[0, pl.ds(b * _NQ + q, 1), :] = (row - keyoff) * inv
            return c2

        return lax.fori_loop(0, _NQ, gq, carry)

    lax.fori_loop(0, _B, gb, 0)


@jax.jit
def kernel(x, batch, projections, cum_weights):
    batch32 = batch.astype(jnp.int32)

    row_idx = pl.pallas_call(
        _prep_body,
        grid=(1,),
        in_specs=[
            pl.BlockSpec((_N // 128, 128), lambda g: (0, 0)),
            pl.BlockSpec((1, _NQ), lambda g: (0, 0)),
        ],
        out_specs=pl.BlockSpec((_B, _NQ), lambda g: (0, 0)),
        out_shape=jax.ShapeDtypeStruct((_B, _NQ), jnp.int32),
    )(batch32.reshape(_N // 128, 128), cum_weights.reshape(1, _NQ))

    rb = 4096
    y = pl.pallas_call(
        _matmul_body,
        grid=(_IT, _N // rb),
        in_specs=[
            pl.BlockSpec((rb, _F), lambda i, r: (r, i)),
            pl.BlockSpec((_F, _NP), lambda i, r: (0, 0)),
            pl.BlockSpec((rb, 1), lambda i, r: (r, 0)),
        ],
        out_specs=pl.BlockSpec((1, rb, _NP), lambda i, r: (i, r, 0)),
        out_shape=jax.ShapeDtypeStruct((_IT, _N, _NP), jnp.float32),
    )(x, projections, batch32.reshape(_N, 1))

    q = pl.pallas_call(
        _sort_body,
        grid=(_IT,),
        in_specs=[
            pl.BlockSpec(memory_space=pl.ANY),
            pl.BlockSpec(memory_space=pltpu.SMEM),
        ],
        out_specs=pl.BlockSpec((1, _B * _NQ, _NP), lambda i: (i, 0, 0)),
        out_shape=jax.ShapeDtypeStruct((_IT, _B * _NQ, _NP), jnp.float32),
        scratch_shapes=[
            pltpu.VMEM((_N, _NP), jnp.float32),
            pltpu.SemaphoreType.DMA,
        ],
    )(y, row_idx)

    out = q.reshape(_IT, _B, _NQ, _NP).transpose(1, 0, 2, 3)
    return out.reshape(_B, _IT * _NQ * _NP)


# speedup vs baseline: 13.5905x; 1.5190x over previous
import numpy as np

import jax
import jax.numpy as jnp
from jax import lax
from jax.experimental import pallas as pl
from jax.experimental.pallas import tpu as pltpu

_N = 32768
_B = 16
_F = 128
_NP = 128
_NQ = 64
_IT = 4
_KEY = 1024.0
_LOGN = 15
_CHN = 512
_LOGC = 9

def _bitmasks(rows):
    r = lax.broadcasted_iota(jnp.int32, (rows, 1), 0)
    return r, [((r >> j) & 1) == 0 for j in range(3)]


def _prep_body(b_ref, cw_ref, idx_ref):
    bv = b_ref[...]
    cw = cw_ref[...]
    off = jnp.int32(0)
    for b in range(_B):
        cnt = jnp.sum((bv == b).astype(jnp.int32))
        cf = (cnt - 1).astype(jnp.float32)
        q = jnp.floor(cw * cf).astype(jnp.int32)
        row = off + q
        row = jnp.where(row < 0, row + _N, row)
        row = jnp.clip(row, 0, _N - 1)
        idx_ref[b, :] = row[0]
        off = off + cnt


def _matmul_body(x_ref, p_ref, b_ref, y_ref):
    acc = jnp.dot(x_ref[...], p_ref[...], preferred_element_type=jnp.float32)
    key = b_ref[...].astype(jnp.float32) * _KEY
    y_ref[0] = acc + key


def _ins(t, pos):
    low = t & ((1 << pos) - 1)
    return ((t >> pos) << (pos + 1)) | low


def _cx_u(yc, j, asc, m3):
    d = 1 << j
    nrows = yc.shape[0]
    if j >= 3:
        t = yc.reshape(nrows // (2 * d), 2, d, _NP)
        lo = t[:, 0]
        hi = t[:, 1]
        mn = jnp.minimum(lo, hi)
        mx = jnp.maximum(lo, hi)
        pair = (mn, mx) if asc else (mx, mn)
        return jnp.stack(pair, axis=1).reshape(nrows, _NP)
    mask = m3[j]
    up = pltpu.roll(yc, nrows - d, 0)
    dn = pltpu.roll(yc, d, 0)
    partner = jnp.where(mask, up, dn)
    if asc:
        return jnp.where(
            mask, jnp.minimum(yc, partner), jnp.maximum(yc, partner)
        )
    return jnp.where(mask, jnp.maximum(yc, partner), jnp.minimum(yc, partner))


def _cx_tiny(yc, kk, j, r, m3):
    d = 1 << j
    take_min = ((r >> j) & 1) == ((r >> kk) & 1)
    up = pltpu.roll(yc, _CHN - d, 0)
    dn = pltpu.roll(yc, d, 0)
    partner = jnp.where(m3[j], up, dn)
    return jnp.where(
        take_min, jnp.minimum(yc, partner), jnp.maximum(yc, partner)
    )


def _sort_body(y_hbm, idx_ref, out_ref, scr_ref, sem):
    cp = pltpu.make_async_copy(y_hbm.at[pl.program_id(0)], scr_ref, sem)
    cp.start()
    cp.wait()

    nch = _N // _CHN

    def chunk_sort(par):
        def body(t, carry):
            ch = t * 2 + par
            a = ch * _CHN
            r512, m512 = _bitmasks(_CHN)
            _, m256 = _bitmasks(_CHN // 2)
            yc = scr_ref[pl.ds(a, _CHN), :]
            for kk in range(1, _LOGC):
                if kk <= 2:
                    for j in range(kk - 1, -1, -1):
                        yc = _cx_tiny(yc, kk, j, r512, m512)
                else:
                    k = 1 << kk
                    t4 = yc.reshape(_CHN // (2 * k), 2, k, _NP)
                    ap = t4[:, 0].reshape(-1, _NP)
                    dp = t4[:, 1].reshape(-1, _NP)
                    for j in range(kk - 1, -1, -1):
                        ap = _cx_u(ap, j, True, m256)
                        dp = _cx_u(dp, j, False, m256)
                    yc = jnp.stack(
                        [
                            ap.reshape(_CHN // (2 * k), k, _NP),
                            dp.reshape(_CHN // (2 * k), k, _NP),
                        ],
                        axis=1,
                    ).reshape(_CHN, _NP)
            for j in range(_LOGC - 1, -1, -1):
                yc = _cx_u(yc, j, par == 0, m512)
            scr_ref[pl.ds(a, _CHN), :] = yc
            return carry

        lax.fori_loop(0, nch // 2, body, 0)

    chunk_sort(0)
    chunk_sort(1)

    for kk in range(_LOGC + 1, _LOGN + 1):
        kc = kk - _LOGC
        js = list(range(kk - 1, _LOGC - 1, -1))
        fused = []
        while js:
            fused.append(tuple(js[:2]))
            js = js[2:]

        for asc in (True, False):
            if not asc and kc >= 6:
                continue
            dirbit = 0 if asc else 1

            for grp in fused:
                if len(grp) == 1:
                    (j,) = grp
                    d = 1 << j
                    jc = j - _LOGC

                    def big(t, carry, d=d, jc=jc, kc=kc, dirbit=dirbit, asc=asc):
                        ca = _ins(_ins(t, jc), kc) | (dirbit << kc)
                        a_lo = ca * _CHN
                        a_hi = a_lo + d
                        ya = scr_ref[pl.ds(a_lo, _CHN), :]
                        yb = scr_ref[pl.ds(a_hi, _CHN), :]
                        mn = jnp.minimum(ya, yb)
                        mx = jnp.maximum(ya, yb)
                        scr_ref[pl.ds(a_lo, _CHN), :] = mn if asc else mx
                        scr_ref[pl.ds(a_hi, _CHN), :] = mx if asc else mn
                        return carry

                    trips = nch // 2 if kc >= 6 else nch // 4
                    lax.fori_loop(0, trips, big, 0)
                else:
                    j1, j2 = grp
                    d1, d2 = 1 << j1, 1 << j2
                    jc2 = j2 - _LOGC

                    def big2(t, carry, d1=d1, d2=d2, jc2=jc2, kc=kc,
                             dirbit=dirbit, asc=asc):
                        ca = _ins(_ins(_ins(t, jc2), jc2 + 1), kc)
                        ca = ca | (dirbit << kc)
                        a0 = ca * _CHN
                        ya = scr_ref[pl.ds(a0, _CHN), :]
                        yb = scr_ref[pl.ds(a0 + d2, _CHN), :]
                        yc = scr_ref[pl.ds(a0 + d1, _CHN), :]
                        yd = scr_ref[pl.ds(a0 + d1 + d2, _CHN), :]
                        mn, mx = jnp.minimum(ya, yc), jnp.maximum(ya, yc)
                        ya, yc = (mn, mx) if asc else (mx, mn)
                        mn, mx = jnp.minimum(yb, yd), jnp.maximum(yb, yd)
                        yb, yd = (mn, mx) if asc else (mx, mn)
                        mn, mx = jnp.minimum(ya, yb), jnp.maximum(ya, yb)
                        ya, yb = (mn, mx) if asc else (mx, mn)
                        mn, mx = jnp.minimum(yc, yd), jnp.maximum(yc, yd)
                        yc, yd = (mn, mx) if asc else (mx, mn)
                        scr_ref[pl.ds(a0, _CHN), :] = ya
                        scr_ref[pl.ds(a0 + d2, _CHN), :] = yb
                        scr_ref[pl.ds(a0 + d1, _CHN), :] = yc
                        scr_ref[pl.ds(a0 + d1 + d2, _CHN), :] = yd
                        return carry

                    trips = nch // 4 if kc >= 6 else nch // 8
                    lax.fori_loop(0, trips, big2, 0)

            def merge_chunk(t, carry, kc=kc, dirbit=dirbit, asc=asc):
                ch = _ins(t, kc) | (dirbit << kc)
                a = ch * _CHN
                _, m512 = _bitmasks(_CHN)
                yc = scr_ref[pl.ds(a, _CHN), :]
                for j in range(_LOGC - 1, -1, -1):
                    yc = _cx_u(yc, j, asc, m512)
                scr_ref[pl.ds(a, _CHN), :] = yc
                return carry

            trips = nch if kc >= 6 else nch // 2
            lax.fori_loop(0, trips, merge_chunk, 0)

    inv = 1.0 / float((_NQ * _NP) ** 0.5)

    def gb(b, carry):
        keyoff = b.astype(jnp.float32) * _KEY

        def gq(q, c2):
            r = idx_ref[b, q]
            row = scr_ref[pl.ds(r, 1), :]
            out_ref[0, pl.ds(b * _NQ + q, 1), :] = (row - keyoff) * inv
            return c2

        return lax.fori_loop(0, _NQ, gq, carry)

    lax.fori_loop(0, _B, gb, 0)


@jax.jit
def kernel(x, batch, projections, cum_weights):
    batch32 = batch.astype(jnp.int32)

    row_idx = pl.pallas_call(
        _prep_body,
        grid=(1,),
        in_specs=[
            pl.BlockSpec((_N // 128, 128), lambda g: (0, 0)),
            pl.BlockSpec((1, _NQ), lambda g: (0, 0)),
        ],
        out_specs=pl.BlockSpec((_B, _NQ), lambda g: (0, 0)),
        out_shape=jax.ShapeDtypeStruct((_B, _NQ), jnp.int32),
    )(batch32.reshape(_N // 128, 128), cum_weights.reshape(1, _NQ))

    rb = 4096
    y = pl.pallas_call(
        _matmul_body,
        grid=(_IT, _N // rb),
        in_specs=[
            pl.BlockSpec((rb, _F), lambda i, r: (r, i)),
            pl.BlockSpec((_F, _NP), lambda i, r: (0, 0)),
            pl.BlockSpec((rb, 1), lambda i, r: (r, 0)),
        ],
        out_specs=pl.BlockSpec((1, rb, _NP), lambda i, r: (i, r, 0)),
        out_shape=jax.ShapeDtypeStruct((_IT, _N, _NP), jnp.float32),
    )(x, projections, batch32.reshape(_N, 1))

    q = pl.pallas_call(
        _sort_body,
        grid=(_IT,),
        in_specs=[
            pl.BlockSpec(memory_space=pl.ANY),
            pl.BlockSpec(memory_space=pltpu.SMEM),
        ],
        out_specs=pl.BlockSpec((1, _B * _NQ, _NP), lambda i: (i, 0, 0)),
        out_shape=jax.ShapeDtypeStruct((_IT, _B * _NQ, _NP), jnp.float32),
        scratch_shapes=[
            pltpu.VMEM((_N, _NP), jnp.float32),
            pltpu.SemaphoreType.DMA,
        ],
    )(y, row_idx)

    out = q.reshape(_IT, _B, _NQ, _NP).transpose(1, 0, 2, 3)
    return out.reshape(_B, _IT * _NQ * _NP)
